# Initial kernel scaffold; baseline (speedup 1.0000x reference)
import jax
import jax.numpy as jnp
from jax.experimental import pallas as pl


def kernel(x_user, x_item, edge_index_ui, edge_index_iu, ew_ui, ew_iu, W_nbr_ui, W_self_ui, b_ui, W_nbr_iu, W_self_iu, b_iu):
    raise NotImplementedError("write your pallas kernel here")


# speedup vs baseline: 3.9680x; 3.9680x over previous
import functools

import jax
import jax.numpy as jnp
from jax import lax
from jax.experimental import pallas as pl
from jax.experimental.pallas import tpu as pltpu
from jax.experimental.pallas import tpu_sc as plsc

N = 10000
D = 128
DH = D // 2
E = 320000
NC = 2
NS = 16
CHUNK = 80
EPT = E // NS
NCHUNK = EPT // CHUNK
ROWS_PT = 624
TAIL0 = NS * ROWS_PT
TAIL = N - TAIL0
RB = 1000


def _tc_body(x_ref, wn_ref, ws_ref, b_ref, y0_ref, y1_ref, b0_ref, b1_ref):
    x = x_ref[0]
    y = jnp.dot(x, wn_ref[0], preferred_element_type=jnp.float32)
    y0_ref[0] = y[:, :DH]
    y1_ref[0] = y[:, DH:]
    base = jnp.dot(x, ws_ref[0], preferred_element_type=jnp.float32) + b_ref[0]
    b0_ref[0] = base[:, :DH]
    b1_ref[0] = base[:, DH:]


def _tc_dense(x_all, wn_all, ws_all, b_all):
    half = jax.ShapeDtypeStruct((2, N, DH), jnp.float32)
    return pl.pallas_call(
        _tc_body,
        grid=(2, N // RB),
        in_specs=[
            pl.BlockSpec((1, RB, D), lambda g, r: (g, r, 0)),
            pl.BlockSpec((1, D, D), lambda g, r: (g, 0, 0)),
            pl.BlockSpec((1, D, D), lambda g, r: (g, 0, 0)),
            pl.BlockSpec((1, 1, D), lambda g, r: (g, 0, 0)),
        ],
        out_specs=[
            pl.BlockSpec((1, RB, DH), lambda g, r: (g, r, 0)),
            pl.BlockSpec((1, RB, DH), lambda g, r: (g, r, 0)),
            pl.BlockSpec((1, RB, DH), lambda g, r: (1 - g, r, 0)),
            pl.BlockSpec((1, RB, DH), lambda g, r: (1 - g, r, 0)),
        ],
        out_shape=[half, half, half, half],
    )(x_all, wn_all, ws_all, b_all)


def _sc_body(y0_hbm, y1_hbm, b0_hbm, b1_hbm, src_hbm, dst_hbm, ew_hbm,
             o0_hbm, o1_hbm, idx_src, idx_dst, ew_v, rows, acc, sem):
    c = lax.axis_index("c")
    s = lax.axis_index("s")
    w = c * NS + s
    row0 = c * N + s * ROWS_PT

    pltpu.sync_copy(src_hbm.at[w], idx_src)
    pltpu.sync_copy(dst_hbm.at[w], idx_dst)
    pltpu.sync_copy(ew_hbm.at[w], ew_v)

    for y_hbm, b_hbm, o_hbm in ((y0_hbm, b0_hbm, o0_hbm),
                                (y1_hbm, b1_hbm, o1_hbm)):
        pltpu.sync_copy(b_hbm.at[pl.ds(row0, ROWS_PT)],
                        acc.at[pl.ds(s * ROWS_PT, ROWS_PT)])

        @pl.when(s == NS - 1)
        def _init_tail():
            pltpu.sync_copy(b_hbm.at[pl.ds(c * N + TAIL0, TAIL)],
                            acc.at[pl.ds(TAIL0, TAIL)])

        plsc.subcore_barrier()

        def chunk(j, carry):
            pltpu.async_copy(y_hbm.at[idx_src.at[j]], rows, sem).wait()
            for g in range(CHUNK // 16):
                w16 = ew_v[pl.ds(j * CHUNK + g * 16, 16)]
                for l in range(16):
                    e = g * 16 + l
                    wspl = w16.at[jnp.full((16,), l, jnp.int32)].get(
                        mode="promise_in_bounds")
                    for d in range(DH // 16):
                        sl = pl.ds(d * 16, 16)
                        rows[e, sl] = rows[e, sl] * wspl
            pltpu.sync_copy(rows, acc.at[idx_dst.at[j]], add=True)
            return carry

        lax.fori_loop(0, NCHUNK, chunk, 0)
        plsc.subcore_barrier()

        pltpu.sync_copy(acc.at[pl.ds(s * ROWS_PT, ROWS_PT)],
                        o_hbm.at[pl.ds(row0, ROWS_PT)])

        @pl.when(s == NS - 1)
        def _write_tail():
            pltpu.sync_copy(acc.at[pl.ds(TAIL0, TAIL)],
                            o_hbm.at[pl.ds(c * N + TAIL0, TAIL)])

        plsc.subcore_barrier()


_sc_agg = functools.partial(
    pl.kernel,
    out_type=[jax.ShapeDtypeStruct((2 * N, DH), jnp.float32),
              jax.ShapeDtypeStruct((2 * N, DH), jnp.float32)],
    mesh=plsc.VectorSubcoreMesh(
        core_axis_name="c", subcore_axis_name="s", num_cores=NC,
        num_subcores=NS),
    compiler_params=pltpu.CompilerParams(use_tc_tiling_on_sc=False),
    scratch_types=[
        pltpu.VMEM((NCHUNK, CHUNK), jnp.int32),
        pltpu.VMEM((NCHUNK, CHUNK), jnp.int32),
        pltpu.VMEM((EPT,), jnp.float32),
        pltpu.VMEM((CHUNK, DH), jnp.float32),
        pltpu.VMEM_SHARED((N, DH), jnp.float32),
        pltpu.SemaphoreType.DMA,
    ],
)(_sc_body)


def kernel(x_user, x_item, edge_index_ui, edge_index_iu, ew_ui, ew_iu,
           W_nbr_ui, W_self_ui, b_ui, W_nbr_iu, W_self_iu, b_iu):
    x_all = jnp.stack([x_user, x_item])
    wn_all = jnp.stack([W_nbr_ui, W_nbr_iu])
    ws_all = jnp.stack([W_self_iu, W_self_ui])
    b_all = jnp.stack([b_iu, b_ui])[:, None, :]
    y0, y1, base0, base1 = _tc_dense(x_all, wn_all, ws_all, b_all)
    y0 = y0.reshape(2 * N, DH)
    y1 = y1.reshape(2 * N, DH)
    base0 = base0.reshape(2 * N, DH)
    base1 = base1.reshape(2 * N, DH)

    src_ui = edge_index_ui[0].astype(jnp.int32).reshape(NS, NCHUNK, CHUNK)
    dst_ui = edge_index_ui[1].astype(jnp.int32).reshape(NS, NCHUNK, CHUNK)
    src_iu = (edge_index_iu[0].astype(jnp.int32) + N).reshape(NS, NCHUNK, CHUNK)
    dst_iu = edge_index_iu[1].astype(jnp.int32).reshape(NS, NCHUNK, CHUNK)
    src3 = jnp.concatenate([src_ui, src_iu], axis=0)
    dst3 = jnp.concatenate([dst_ui, dst_iu], axis=0)
    ew3 = jnp.concatenate([ew_ui.reshape(NS, EPT),
                           ew_iu.reshape(NS, EPT)], axis=0)

    out0, out1 = _sc_agg(y0, y1, base0, base1, src3, dst3, ew3)
    out_cat = jnp.concatenate([out0, out1], axis=1)
    out_item = out_cat[:N]
    out_user = out_cat[N:]
    return (out_user, out_item)
